# Initial kernel scaffold; baseline (speedup 1.0000x reference)
import functools

import jax
import jax.numpy as jnp
from jax import lax
from jax.experimental import pallas as pl
from jax.experimental.pallas import tpu as pltpu
from jax.experimental.pallas import tpu_sc as plsc

D = 128


def _proj_body(t_ref, w_ref, b_ref, o_ref):
    o_ref[...] = lax.dot_general(
        jnp.maximum(t_ref[...], 0.0), w_ref[...],
        dimension_numbers=(((1,), (1,)), ((), ())),
        preferred_element_type=jnp.float32,
    ) + b_ref[...]


def _proj_table(table, W, b):
    V = table.shape[0]
    RB = 512
    return pl.pallas_call(
        _proj_body,
        grid=(pl.cdiv(V, RB),),
        in_specs=[
            pl.BlockSpec((RB, D), lambda i: (i, 0)),
            pl.BlockSpec((D, D), lambda i: (0, 0)),
            pl.BlockSpec((1, D), lambda i: (0, 0)),
        ],
        out_specs=pl.BlockSpec((RB, D), lambda i: (i, 0)),
        out_shape=jax.ShapeDtypeStruct((V, D), jnp.float32),
    )(table, W, b.reshape(1, D))


_INFO = plsc.get_sparse_core_info()
_NC, _NS = _INFO.num_cores, _INFO.num_subcores
_NW = _NC * _NS


@functools.lru_cache(maxsize=None)
def _make_gather(V, counts):
    max_rows = max(counts)
    mesh = plsc.VectorSubcoreMesh(core_axis_name="c", subcore_axis_name="s")

    out_type = tuple(
        jax.ShapeDtypeStruct((c * _NW * 128, D), jnp.float32) for c in counts
    ) * 2

    @functools.partial(
        pl.kernel,
        out_type=out_type,
        mesh=mesh,
        scratch_types=[
            pltpu.VMEM((max_rows, 128), jnp.int32),
            pltpu.VMEM((128, D), jnp.float32),
            pltpu.SemaphoreType.DMA,
        ],
    )
    def gather(table_hbm, proj_hbm, i0, i1, i2, i3,
               e0, e1, e2, e3, p0, p1, p2, p3,
               idx_v, rows_v, sem):
        wid = lax.axis_index("s") * _NC + lax.axis_index("c")
        idx_refs = (i0, i1, i2, i3)
        emb_outs = (e0, e1, e2, e3)
        proj_outs = (p0, p1, p2, p3)
        for seg in range(4):
            nr = counts[seg]
            base_r = wid * nr
            pltpu.sync_copy(idx_refs[seg].at[pl.ds(base_r, nr)],
                            idx_v.at[pl.ds(0, nr)])
            for tbl, out in ((table_hbm, emb_outs[seg]),
                             (proj_hbm, proj_outs[seg])):
                def body(j, _, tbl=tbl, out=out, base_r=base_r):
                    pltpu.async_copy(tbl.at[idx_v.at[j]], rows_v, sem).wait()
                    pltpu.sync_copy(
                        rows_v, out.at[pl.ds((base_r + j) * 128, 128)])
                    return 0
                lax.fori_loop(0, nr, body, 0)

    return gather


def kernel(tensor_demo, tensor_med, tensor_vitals, tensor_labs, table, W, b):
    V = table.shape[0]
    proj_tab = _proj_table(table, W, b)

    idxs = []
    shapes = []
    counts = []
    for t in (tensor_demo, tensor_med, tensor_vitals, tensor_labs):
        shapes.append(t.shape)
        n = t.shape[0] * t.shape[1]
        counts.append(n // (128 * _NW))
        idxs.append(t.astype(jnp.int32).reshape(n // 128, 128))

    outs = _make_gather(V, tuple(counts))(table, proj_tab, *idxs)
    embs = tuple(o.reshape(s[0], s[1], D) for o, s in zip(outs[:4], shapes))
    projs = tuple(o.reshape(s[0], s[1], D) for o, s in zip(outs[4:], shapes))
    return (embs, projs, embs, projs)


# speedup vs baseline: 1.4799x; 1.4799x over previous
import functools

import jax
import jax.numpy as jnp
from jax import lax
from jax.experimental import pallas as pl
from jax.experimental.pallas import tpu as pltpu
from jax.experimental.pallas import tpu_sc as plsc

D = 128


def _proj_body(t_ref, w_ref, b_ref, o_ref):
    o_ref[...] = lax.dot_general(
        jnp.maximum(t_ref[...], 0.0), w_ref[...],
        dimension_numbers=(((1,), (1,)), ((), ())),
        preferred_element_type=jnp.float32,
    ) + b_ref[...]


def _proj_table(table, W, b):
    V = table.shape[0]
    RB = 512
    return pl.pallas_call(
        _proj_body,
        grid=(pl.cdiv(V, RB),),
        in_specs=[
            pl.BlockSpec((RB, D), lambda i: (i, 0)),
            pl.BlockSpec((D, D), lambda i: (0, 0)),
            pl.BlockSpec((1, D), lambda i: (0, 0)),
        ],
        out_specs=pl.BlockSpec((RB, D), lambda i: (i, 0)),
        out_shape=jax.ShapeDtypeStruct((V, D), jnp.float32),
    )(table, W, b.reshape(1, D))


_INFO = plsc.get_sparse_core_info()
_NC, _NS = _INFO.num_cores, _INFO.num_subcores
_NW = _NC * _NS


@functools.lru_cache(maxsize=None)
def _make_gather(V, counts):
    max_rows = max(counts)
    mesh = plsc.VectorSubcoreMesh(core_axis_name="c", subcore_axis_name="s")

    out_type = tuple(
        jax.ShapeDtypeStruct((c * _NW * 128, D), jnp.float32) for c in counts
    ) * 2

    @functools.partial(
        pl.kernel,
        out_type=out_type,
        mesh=mesh,
        compiler_params=pltpu.CompilerParams(use_tc_tiling_on_sc=False),
        scratch_types=[
            pltpu.VMEM((max_rows, 128), jnp.int32),
            pltpu.VMEM((128, D), jnp.float32),
            pltpu.SemaphoreType.DMA,
        ],
    )
    def gather(table_hbm, proj_hbm, i0, i1, i2, i3,
               e0, e1, e2, e3, p0, p1, p2, p3,
               idx_v, rows_v, sem):
        wid = lax.axis_index("s") * _NC + lax.axis_index("c")
        idx_refs = (i0, i1, i2, i3)
        emb_outs = (e0, e1, e2, e3)
        proj_outs = (p0, p1, p2, p3)
        for seg in range(4):
            nr = counts[seg]
            base_r = wid * nr
            pltpu.sync_copy(idx_refs[seg].at[pl.ds(base_r, nr)],
                            idx_v.at[pl.ds(0, nr)])
            for tbl, out in ((table_hbm, emb_outs[seg]),
                             (proj_hbm, proj_outs[seg])):
                def body(j, _, tbl=tbl, out=out, base_r=base_r):
                    pltpu.async_copy(tbl.at[idx_v.at[j]], rows_v, sem).wait()
                    pltpu.sync_copy(
                        rows_v, out.at[pl.ds((base_r + j) * 128, 128)])
                    return 0
                lax.fori_loop(0, nr, body, 0)

    return gather


def kernel(tensor_demo, tensor_med, tensor_vitals, tensor_labs, table, W, b):
    V = table.shape[0]
    proj_tab = _proj_table(table, W, b)

    idxs = []
    shapes = []
    counts = []
    for t in (tensor_demo, tensor_med, tensor_vitals, tensor_labs):
        shapes.append(t.shape)
        n = t.shape[0] * t.shape[1]
        counts.append(n // (128 * _NW))
        idxs.append(t.astype(jnp.int32).reshape(n // 128, 128))

    outs = _make_gather(V, tuple(counts))(table, proj_tab, *idxs)
    embs = tuple(o.reshape(s[0], s[1], D) for o, s in zip(outs[:4], shapes))
    projs = tuple(o.reshape(s[0], s[1], D) for o, s in zip(outs[4:], shapes))
    return (embs, projs, embs, projs)


# speedup vs baseline: 1.6588x; 1.1208x over previous
import functools

import jax
import jax.numpy as jnp
from jax import lax
from jax.experimental import pallas as pl
from jax.experimental.pallas import tpu as pltpu
from jax.experimental.pallas import tpu_sc as plsc

D = 128


def _proj_body(t_ref, w_ref, b_ref, o_ref):
    o_ref[...] = lax.dot_general(
        jnp.maximum(t_ref[...], 0.0), w_ref[...],
        dimension_numbers=(((1,), (1,)), ((), ())),
        preferred_element_type=jnp.float32,
    ) + b_ref[...]


def _proj_table(table, W, b):
    V = table.shape[0]
    RB = 512
    return pl.pallas_call(
        _proj_body,
        grid=(pl.cdiv(V, RB),),
        in_specs=[
            pl.BlockSpec((RB, D), lambda i: (i, 0)),
            pl.BlockSpec((D, D), lambda i: (0, 0)),
            pl.BlockSpec((1, D), lambda i: (0, 0)),
        ],
        out_specs=pl.BlockSpec((RB, D), lambda i: (i, 0)),
        out_shape=jax.ShapeDtypeStruct((V, D), jnp.float32),
    )(table, W, b.reshape(1, D))


_INFO = plsc.get_sparse_core_info()
_NC, _NS = _INFO.num_cores, _INFO.num_subcores
_NW = _NC * _NS


_NBUF = 4


@functools.lru_cache(maxsize=None)
def _make_gather(V, counts):
    max_rows = max(counts)
    mesh = plsc.VectorSubcoreMesh(core_axis_name="c", subcore_axis_name="s")

    out_type = tuple(
        jax.ShapeDtypeStruct((c * _NW * 128, D), jnp.float32) for c in counts
    ) * 2

    @functools.partial(
        pl.kernel,
        out_type=out_type,
        mesh=mesh,
        compiler_params=pltpu.CompilerParams(use_tc_tiling_on_sc=False),
        scratch_types=[pltpu.VMEM((max_rows, 128), jnp.int32)]
        + [pltpu.VMEM((128, D), jnp.float32) for _ in range(_NBUF)]
        + [pltpu.SemaphoreType.DMA for _ in range(_NBUF)],
    )
    def gather(table_hbm, proj_hbm, i0, i1, i2, i3,
               e0, e1, e2, e3, p0, p1, p2, p3,
               idx_v, *bufs_sems):
        bufs = bufs_sems[:_NBUF]
        sems = bufs_sems[_NBUF:]
        wid = lax.axis_index("s") * _NC + lax.axis_index("c")
        idx_refs = (i0, i1, i2, i3)
        emb_outs = (e0, e1, e2, e3)
        proj_outs = (p0, p1, p2, p3)

        def pipeline(tbl, out, nr, base_r):
            def issue(j, b):
                pltpu.async_copy(tbl.at[idx_v.at[j]], bufs[b], sems[b])

            def retire(j, b):
                pltpu.make_async_copy(
                    tbl.at[pl.ds(0, 128)], bufs[b], sems[b]).wait()
                pltpu.sync_copy(
                    bufs[b], out.at[pl.ds((base_r + j) * 128, 128)])

            for b in range(_NBUF):
                issue(b, b)
            steady = nr - _NBUF
            nk = (steady + _NBUF - 1) // _NBUF

            def body(k, _):
                for b in range(_NBUF):
                    j = k * _NBUF + b

                    @pl.when(j < steady)
                    def _(j=j, b=b):
                        retire(j, b)
                        issue(j + _NBUF, b)
                return 0

            lax.fori_loop(0, nk, body, 0)
            for j in range(max(steady, 0), nr):
                retire(j, j % _NBUF)

        for seg in range(4):
            nr = counts[seg]
            base_r = wid * nr
            pltpu.sync_copy(idx_refs[seg].at[pl.ds(base_r, nr)],
                            idx_v.at[pl.ds(0, nr)])
            pipeline(table_hbm, emb_outs[seg], nr, base_r)
            pipeline(proj_hbm, proj_outs[seg], nr, base_r)

    return gather


def kernel(tensor_demo, tensor_med, tensor_vitals, tensor_labs, table, W, b):
    V = table.shape[0]
    proj_tab = _proj_table(table, W, b)

    idxs = []
    shapes = []
    counts = []
    for t in (tensor_demo, tensor_med, tensor_vitals, tensor_labs):
        shapes.append(t.shape)
        n = t.shape[0] * t.shape[1]
        counts.append(n // (128 * _NW))
        idxs.append(t.astype(jnp.int32).reshape(n // 128, 128))

    outs = _make_gather(V, tuple(counts))(table, proj_tab, *idxs)
    embs = tuple(o.reshape(s[0], s[1], D) for o, s in zip(outs[:4], shapes))
    projs = tuple(o.reshape(s[0], s[1], D) for o, s in zip(outs[4:], shapes))
    return (embs, projs, embs, projs)
